# baseline (speedup 1.0000x reference)
import functools

import jax
import jax.numpy as jnp
from jax import lax
from jax.experimental import pallas as pl
from jax.experimental.pallas import tpu as pltpu
from jax.experimental.pallas import tpu_sc as plsc

VOCAB = 1_000_000
EMBED_D = 64
BATCH = 4096
SEQ = 200
NUM_CORES = 2
NUM_SUBCORES = 16
D_PER_CORE = EMBED_D // NUM_CORES
B_PER_TILE = BATCH // NUM_SUBCORES
BP = 128
NPASS = B_PER_TILE // BP

_mesh = plsc.VectorSubcoreMesh(core_axis_name="c", subcore_axis_name="s")


@functools.partial(
    pl.kernel,
    mesh=_mesh,
    out_type=jax.ShapeDtypeStruct((SEQ, EMBED_D * BATCH), jnp.float32),
    scratch_types=[
        [pltpu.VMEM((SEQ, BP), jnp.int32) for _ in range(NPASS)],
        [pltpu.VMEM((SEQ, BP), jnp.float32) for _ in range(NPASS)],
        [pltpu.SemaphoreType.DMA for _ in range(NPASS)],
        [pltpu.SemaphoreType.DMA for _ in range(NPASS)],
    ],
    compiler_params=pltpu.CompilerParams(use_tc_tiling_on_sc=False),
)
def _dgather(table_t, x_t, out_hbm, idx_t, val, gsem, wsem):
    cid = lax.axis_index("c")
    sid = lax.axis_index("s")
    b0 = sid * B_PER_TILE

    for p in range(NPASS):
        pltpu.sync_copy(x_t.at[:, pl.ds(b0 + p * BP, BP)], idx_t[p])

    def dbody(k, _):
        dg = cid * D_PER_CORE + k
        for p in range(NPASS):
            @pl.when(k > 0)
            def _():
                prev = out_hbm.at[
                    :, pl.ds((dg - 1) * BATCH + b0 + p * BP, BP)
                ]
                pltpu.make_async_copy(val[p], prev, wsem[p]).wait()

            def grow(s, _):
                pltpu.make_async_copy(
                    table_t.at[dg].at[idx_t[p].at[s]], val[p].at[s], gsem[p]
                ).start()
                return ()

            lax.fori_loop(0, SEQ, grow, ())

        for p in range(NPASS):
            out_slab = out_hbm.at[:, pl.ds(dg * BATCH + b0 + p * BP, BP)]
            pltpu.make_async_copy(out_slab, val[p], gsem[p]).wait()
            pltpu.make_async_copy(val[p], out_slab, wsem[p]).start()

        return ()

    lax.fori_loop(0, D_PER_CORE, dbody, ())
    dlast = cid * D_PER_CORE + D_PER_CORE - 1
    for p in range(NPASS):
        last = out_hbm.at[:, pl.ds(dlast * BATCH + b0 + p * BP, BP)]
        pltpu.make_async_copy(val[p], last, wsem[p]).wait()


def kernel(x, embeddings):
    out_t = _dgather(embeddings.T, x.T)
    return out_t.reshape(SEQ, EMBED_D, BATCH).transpose(2, 0, 1)


# speedup vs baseline: 5.7523x; 5.7523x over previous
import functools

import jax
import jax.numpy as jnp
from jax import lax
from jax.experimental import pallas as pl
from jax.experimental.pallas import tpu as pltpu
from jax.experimental.pallas import tpu_sc as plsc

EMBED_D = 64
BATCH = 4096
SEQ = 200
NUM_WORKERS = 32
ROWS_PER_W = BATCH // NUM_WORKERS
NB = 4
NGROUP = ROWS_PER_W // NB

_mesh = plsc.VectorSubcoreMesh(core_axis_name="c", subcore_axis_name="s")


@functools.partial(
    pl.kernel,
    mesh=_mesh,
    out_type=jax.ShapeDtypeStruct((BATCH, SEQ, EMBED_D), jnp.float32),
    scratch_types=[
        pltpu.VMEM((ROWS_PER_W, SEQ), jnp.int32),
        [pltpu.VMEM((SEQ, EMBED_D), jnp.float32) for _ in range(NB)],
        [pltpu.SemaphoreType.DMA for _ in range(NB)],
        [pltpu.SemaphoreType.DMA for _ in range(NB)],
    ],
    compiler_params=pltpu.CompilerParams(use_tc_tiling_on_sc=False),
)
def _gather_kernel(table_hbm, x_hbm, out_hbm, idx_v, rows, gsem, ssem):
    wid = lax.axis_index("s") * 2 + lax.axis_index("c")
    base = wid * ROWS_PER_W
    pltpu.sync_copy(x_hbm.at[pl.ds(base, ROWS_PER_W)], idx_v)

    def gather(r, b):
        return pltpu.make_async_copy(table_hbm.at[idx_v.at[r]], rows[b], gsem[b])

    def store(r, b):
        return pltpu.make_async_copy(rows[b], out_hbm.at[base + r], ssem[b])

    def body(p, _):
        r0 = p * NB
        for b in range(NB):
            @pl.when(p > 0)
            def _():
                store(r0 + b - NB, b).wait()
            gather(r0 + b, b).start()
        for b in range(NB):
            gather(r0 + b, b).wait()
            store(r0 + b, b).start()
        return ()

    lax.fori_loop(0, NGROUP, body, ())
    for b in range(NB):
        store(ROWS_PER_W - NB + b, b).wait()


def kernel(x, embeddings):
    return _gather_kernel(embeddings, x)
